# Initial kernel scaffold; baseline (speedup 1.0000x reference)
import jax
import jax.numpy as jnp
from jax.experimental import pallas as pl


def kernel(x, Wc, bc, wealth, W_gate, W_up, W_down, A_gate, B_gate, A_up, B_up, A_down, B_down):
    raise NotImplementedError("write your pallas kernel here")


# speedup vs baseline: 2.6371x; 2.6371x over previous
import functools

import jax
import jax.numpy as jnp
from jax.experimental import pallas as pl

E = 8
TOPK = 2
D = 768
FF = 2048
R = 64
SCALING = 16.0 / 64.0

TBLK = 128


def _ffn_kernel(x_ref, Wc_ref, bc_ref, wealth_ref,
                Wg_ref, Wu_ref, Wd_ref,
                Ag_ref, Bg_ref, Au_ref, Bu_ref, Ad_ref, Bd_ref,
                out_ref):
    xb = x_ref[...]

    logits = jnp.dot(xb, Wc_ref[...].T, preferred_element_type=jnp.float32)
    bids = jax.nn.sigmoid(logits + bc_ref[...]) * wealth_ref[...]
    iota = jax.lax.broadcasted_iota(jnp.int32, (TBLK, E), 1)
    m1 = jnp.max(bids, axis=-1, keepdims=True)
    i1 = jnp.min(jnp.where(bids == m1, iota, E), axis=-1, keepdims=True)
    oh1 = iota == i1
    masked = jnp.where(oh1, -jnp.inf, bids)
    m2 = jnp.max(masked, axis=-1, keepdims=True)
    i2 = jnp.min(jnp.where(masked == m2, iota, E), axis=-1, keepdims=True)
    oh2 = iota == i2
    w1 = jax.nn.sigmoid(m1 - m2)
    W8 = jnp.where(oh1, w1, 0.0) + jnp.where(oh2, 1.0 - w1, 0.0)

    G0 = jnp.dot(xb, Wg_ref[...], preferred_element_type=jnp.float32)
    U0 = jnp.dot(xb, Wu_ref[...], preferred_element_type=jnp.float32)

    H = jnp.zeros((TBLK, FF), dtype=jnp.float32)
    dlora = jnp.zeros((TBLK, D), dtype=jnp.float32)
    for e in range(E):
        dg = jnp.dot(jnp.dot(xb, Ag_ref[e], preferred_element_type=jnp.float32),
                     Bg_ref[e], preferred_element_type=jnp.float32) * SCALING
        du = jnp.dot(jnp.dot(xb, Au_ref[e], preferred_element_type=jnp.float32),
                     Bu_ref[e], preferred_element_type=jnp.float32) * SCALING
        h = jax.nn.silu(G0 + dg) * (U0 + du)
        hw = W8[:, e:e + 1] * h
        H = H + hw
        dlora = dlora + jnp.dot(
            jnp.dot(hw, Ad_ref[e], preferred_element_type=jnp.float32),
            Bd_ref[e], preferred_element_type=jnp.float32)

    out_ref[...] = (jnp.dot(H, Wd_ref[...], preferred_element_type=jnp.float32)
                    + dlora * SCALING)


@functools.partial(jax.jit, static_argnames=("interpret",))
def _run(x2d, Wc, bc, wealth, W_gate, W_up, W_down,
         A_gate, B_gate, A_up, B_up, A_down, B_down, interpret=False):
    S = x2d.shape[0]
    grid = (S // TBLK,)
    full = lambda *shape: pl.BlockSpec(shape, lambda i: (0,) * len(shape))
    return pl.pallas_call(
        _ffn_kernel,
        grid=grid,
        in_specs=[
            pl.BlockSpec((TBLK, D), lambda i: (i, 0)),
            full(E, D),
            full(1, E),
            full(1, E),
            full(D, FF),
            full(D, FF),
            full(FF, D),
            full(E, D, R),
            full(E, R, FF),
            full(E, D, R),
            full(E, R, FF),
            full(E, FF, R),
            full(E, R, D),
        ],
        out_specs=pl.BlockSpec((TBLK, D), lambda i: (i, 0)),
        out_shape=jax.ShapeDtypeStruct((S, D), jnp.float32),
        interpret=interpret,
    )(x2d, Wc, bc.reshape(1, E), wealth.reshape(1, E), W_gate, W_up, W_down,
      A_gate, B_gate, A_up, B_up, A_down, B_down)


def kernel(x, Wc, bc, wealth, W_gate, W_up, W_down,
           A_gate, B_gate, A_up, B_up, A_down, B_down):
    B, S, _ = x.shape
    out = _run(x.reshape(B * S, D), Wc, bc, wealth, W_gate, W_up, W_down,
               A_gate, B_gate, A_up, B_up, A_down, B_down)
    return out.reshape(B, S, D)


# speedup vs baseline: 3.6977x; 1.4022x over previous
import functools

import jax
import jax.numpy as jnp
from jax.experimental import pallas as pl

E = 8
TOPK = 2
D = 768
FF = 2048
R = 64
SCALING = 16.0 / 64.0

TBLK = 128


def _ffn_kernel(x_ref, Wc_ref, bc_ref, wealth_ref,
                Wg_ref, Wu_ref, Wd_ref,
                Agu_ref, Bg_ref, Bu_ref, Ad_ref, Bd2_ref,
                out_ref):
    xb = x_ref[...]

    logits = jnp.dot(xb, Wc_ref[...].T, preferred_element_type=jnp.float32)
    bids = jax.nn.sigmoid(logits + bc_ref[...]) * wealth_ref[...]
    iota = jax.lax.broadcasted_iota(jnp.int32, (TBLK, E), 1)
    m1 = jnp.max(bids, axis=-1, keepdims=True)
    i1 = jnp.min(jnp.where(bids == m1, iota, E), axis=-1, keepdims=True)
    oh1 = iota == i1
    masked = jnp.where(oh1, -jnp.inf, bids)
    m2 = jnp.max(masked, axis=-1, keepdims=True)
    i2 = jnp.min(jnp.where(masked == m2, iota, E), axis=-1, keepdims=True)
    oh2 = iota == i2
    w1 = jax.nn.sigmoid(m1 - m2)
    W8 = jnp.where(oh1, w1, 0.0) + jnp.where(oh2, 1.0 - w1, 0.0)

    G0 = jnp.dot(xb, Wg_ref[...], preferred_element_type=jnp.float32)
    U0 = jnp.dot(xb, Wu_ref[...], preferred_element_type=jnp.float32)

    P = jnp.dot(xb, Agu_ref[...], preferred_element_type=jnp.float32)

    H = jnp.zeros((TBLK, FF), dtype=jnp.float32)
    pds = []
    for e in range(E):
        dg = jnp.dot(P[:, e * R:(e + 1) * R], Bg_ref[e],
                     preferred_element_type=jnp.float32) * SCALING
        du = jnp.dot(P[:, E * R + e * R:E * R + (e + 1) * R], Bu_ref[e],
                     preferred_element_type=jnp.float32) * SCALING
        h = jax.nn.silu(G0 + dg) * (U0 + du)
        hw = W8[:, e:e + 1] * h
        H = H + hw
        pds.append(jnp.dot(hw, Ad_ref[e], preferred_element_type=jnp.float32))

    PD = jnp.concatenate(pds, axis=1)
    dlora = jnp.dot(PD, Bd2_ref[...], preferred_element_type=jnp.float32)

    out_ref[...] = (jnp.dot(H, Wd_ref[...], preferred_element_type=jnp.float32)
                    + dlora * SCALING)


@functools.partial(jax.jit, static_argnames=("interpret",))
def _run(x2d, Wc, bc, wealth, W_gate, W_up, W_down,
         A_gate, B_gate, A_up, B_up, A_down, B_down, interpret=False):
    S = x2d.shape[0]
    grid = (S // TBLK,)
    full = lambda *shape: pl.BlockSpec(shape, lambda i: (0,) * len(shape))
    return pl.pallas_call(
        _ffn_kernel,
        grid=grid,
        in_specs=[
            pl.BlockSpec((TBLK, D), lambda i: (i, 0)),
            full(E, D),
            full(1, E),
            full(1, E),
            full(D, FF),
            full(D, FF),
            full(FF, D),
            full(D, 2 * E * R),
            full(E, R, FF),
            full(E, R, FF),
            full(E, FF, R),
            full(E * R, D),
        ],
        out_specs=pl.BlockSpec((TBLK, D), lambda i: (i, 0)),
        out_shape=jax.ShapeDtypeStruct((S, D), jnp.float32),
        interpret=interpret,
    )(x2d, Wc, bc.reshape(1, E), wealth.reshape(1, E), W_gate, W_up, W_down,
      jnp.concatenate([A_gate.transpose(1, 0, 2).reshape(D, E * R),
                       A_up.transpose(1, 0, 2).reshape(D, E * R)], axis=1),
      B_gate, B_up, A_down, B_down.reshape(E * R, D))


def kernel(x, Wc, bc, wealth, W_gate, W_up, W_down,
           A_gate, B_gate, A_up, B_up, A_down, B_down):
    B, S, _ = x.shape
    out = _run(x.reshape(B * S, D), Wc, bc, wealth, W_gate, W_up, W_down,
               A_gate, B_gate, A_up, B_up, A_down, B_down)
    return out.reshape(B, S, D)


# speedup vs baseline: 4.1190x; 1.1140x over previous
import functools

import jax
import jax.numpy as jnp
from jax.experimental import pallas as pl

E = 8
TOPK = 2
D = 768
FF = 2048
R = 64
SCALING = 16.0 / 64.0

TBLK = 256


def _ffn_kernel(x_ref, Wc_ref, bc_ref, wealth_ref,
                Wg_ref, Wu_ref, Wd_ref,
                Agu_ref, Bg_ref, Bu_ref, Ad_ref, Bd2_ref,
                out_ref):
    xb = x_ref[...]

    logits = jnp.dot(xb, Wc_ref[...].T, preferred_element_type=jnp.float32)
    bids = jax.nn.sigmoid(logits + bc_ref[...]) * wealth_ref[...]
    iota = jax.lax.broadcasted_iota(jnp.int32, (TBLK, E), 1)
    m1 = jnp.max(bids, axis=-1, keepdims=True)
    i1 = jnp.min(jnp.where(bids == m1, iota, E), axis=-1, keepdims=True)
    oh1 = iota == i1
    masked = jnp.where(oh1, -jnp.inf, bids)
    m2 = jnp.max(masked, axis=-1, keepdims=True)
    i2 = jnp.min(jnp.where(masked == m2, iota, E), axis=-1, keepdims=True)
    oh2 = iota == i2
    w1 = jax.nn.sigmoid(m1 - m2)
    W8 = jnp.where(oh1, w1, 0.0) + jnp.where(oh2, 1.0 - w1, 0.0)

    G0 = jnp.dot(xb, Wg_ref[...], preferred_element_type=jnp.float32)
    U0 = jnp.dot(xb, Wu_ref[...], preferred_element_type=jnp.float32)

    P = jnp.dot(xb, Agu_ref[...], preferred_element_type=jnp.float32)

    H = jnp.zeros((TBLK, FF), dtype=jnp.float32)
    pds = []
    for e in range(E):
        dg = jnp.dot(P[:, e * R:(e + 1) * R], Bg_ref[e],
                     preferred_element_type=jnp.float32) * SCALING
        du = jnp.dot(P[:, E * R + e * R:E * R + (e + 1) * R], Bu_ref[e],
                     preferred_element_type=jnp.float32) * SCALING
        h = jax.nn.silu(G0 + dg) * (U0 + du)
        hw = W8[:, e:e + 1] * h
        H = H + hw
        pds.append(jnp.dot(hw, Ad_ref[e], preferred_element_type=jnp.float32))

    PD = jnp.concatenate(pds, axis=1)
    dlora = jnp.dot(PD, Bd2_ref[...], preferred_element_type=jnp.float32)

    out_ref[...] = (jnp.dot(H, Wd_ref[...], preferred_element_type=jnp.float32)
                    + dlora * SCALING)


@functools.partial(jax.jit, static_argnames=("interpret",))
def _run(x2d, Wc, bc, wealth, W_gate, W_up, W_down,
         A_gate, B_gate, A_up, B_up, A_down, B_down, interpret=False):
    S = x2d.shape[0]
    grid = (S // TBLK,)
    full = lambda *shape: pl.BlockSpec(shape, lambda i: (0,) * len(shape))
    return pl.pallas_call(
        _ffn_kernel,
        grid=grid,
        in_specs=[
            pl.BlockSpec((TBLK, D), lambda i: (i, 0)),
            full(E, D),
            full(1, E),
            full(1, E),
            full(D, FF),
            full(D, FF),
            full(FF, D),
            full(D, 2 * E * R),
            full(E, R, FF),
            full(E, R, FF),
            full(E, FF, R),
            full(E * R, D),
        ],
        out_specs=pl.BlockSpec((TBLK, D), lambda i: (i, 0)),
        out_shape=jax.ShapeDtypeStruct((S, D), jnp.float32),
        interpret=interpret,
    )(x2d, Wc, bc.reshape(1, E), wealth.reshape(1, E), W_gate, W_up, W_down,
      jnp.concatenate([A_gate.transpose(1, 0, 2).reshape(D, E * R),
                       A_up.transpose(1, 0, 2).reshape(D, E * R)], axis=1),
      B_gate, B_up, A_down, B_down.reshape(E * R, D))


def kernel(x, Wc, bc, wealth, W_gate, W_up, W_down,
           A_gate, B_gate, A_up, B_up, A_down, B_down):
    B, S, _ = x.shape
    out = _run(x.reshape(B * S, D), Wc, bc, wealth, W_gate, W_up, W_down,
               A_gate, B_gate, A_up, B_up, A_down, B_down)
    return out.reshape(B, S, D)
